# Initial kernel scaffold; baseline (speedup 1.0000x reference)
import jax
import jax.numpy as jnp
from jax.experimental import pallas as pl


def kernel(atom_features, atom_indices):
    raise NotImplementedError("write your pallas kernel here")


# speedup vs baseline: 1.8848x; 1.8848x over previous
import functools

import jax
import jax.numpy as jnp
from jax import lax
from jax.experimental import pallas as pl
from jax.experimental.pallas import tpu as pltpu
from jax.experimental.pallas import tpu_sc as plsc

B = 512
A = 256
D = 128
NC = 2
NS = 16
NW = NC * NS
CPW = B // NW
LANES = 16
NV = D // LANES
IDX_CHUNK = 128
SCALE = 1.0 / A


def _pool_body(table_hbm, idx_hbm, out_hbm, idx_v, buf0, buf1, out_v,
               sem0, sem1):
    wid = lax.axis_index("s") * NC + lax.axis_index("c")
    base = wid * CPW
    pltpu.sync_copy(idx_hbm.at[pl.ds(base * A, CPW * A)], idx_v)

    bufs = (buf0, buf1)
    sems = (sem0, sem1)
    pending = [None] * CPW

    def start(c):
        slot = c % 2
        off = c * A
        cps = []
        for h in range(A // IDX_CHUNK):
            cps.append(pltpu.async_copy(
                table_hbm.at[idx_v.at[pl.ds(off + h * IDX_CHUNK, IDX_CHUNK)]],
                bufs[slot].at[pl.ds(h * IDX_CHUNK, IDX_CHUNK)],
                sems[slot]))
        pending[c] = cps

    start(0)
    for c in range(CPW):
        if c + 1 < CPW:
            start(c + 1)
        for cp in pending[c]:
            cp.wait()
        buf = bufs[c % 2]

        def body(r, acc, buf=buf):
            return tuple(acc[j] + buf[r, pl.ds(j * LANES, LANES)]
                         for j in range(NV))

        acc = lax.fori_loop(
            0, A, body,
            tuple(jnp.zeros((LANES,), jnp.float32) for _ in range(NV)))
        for j in range(NV):
            out_v[c, pl.ds(j * LANES, LANES)] = acc[j] * jnp.float32(SCALE)

    pltpu.sync_copy(out_v, out_hbm.at[pl.ds(base, CPW)])


@functools.partial(jax.jit)
def _pool(table, idx_flat):
    f = pl.kernel(
        _pool_body,
        out_type=jax.ShapeDtypeStruct((B, D), jnp.float32),
        mesh=plsc.VectorSubcoreMesh(core_axis_name="c", subcore_axis_name="s"),
        scratch_types=[
            pltpu.VMEM((CPW * A,), jnp.int32),
            pltpu.VMEM((A, D), jnp.float32),
            pltpu.VMEM((A, D), jnp.float32),
            pltpu.VMEM((CPW, D), jnp.float32),
            pltpu.SemaphoreType.DMA,
            pltpu.SemaphoreType.DMA,
        ],
    )
    return f(table, idx_flat)


def kernel(atom_features, atom_indices):
    idx_flat = atom_indices.reshape(-1).astype(jnp.int32)
    return _pool(atom_features, idx_flat)
